# Initial kernel scaffold; baseline (speedup 1.0000x reference)
import jax
import jax.numpy as jnp
from jax.experimental import pallas as pl


def kernel(x, edge_index, batch, scatter_edge_index, scatter_edge_attr, W_in, b_in, Wc1, bc1, Wc2, bc2, Wf1, bf1, g1, be1, Wf2, bf2, g2, be2):
    raise NotImplementedError("write your pallas kernel here")


# speedup vs baseline: 1.0391x; 1.0391x over previous
import functools

import jax
import jax.numpy as jnp
from jax import lax
from jax.experimental import pallas as pl
from jax.experimental.pallas import tpu as pltpu
from jax.experimental.pallas import tpu_sc as plsc

N = 10000
E = 320000
H = 128
G = 64
C = 10

NC = 2
NS = 16
NW = NC * NS
CB = 64
NB = 160
CBUF = 384
SCH = 8000
BSC1 = 3000
ZPT = 800
PP = 128

_HALF = [10080 * k for k in range(1, 12)] + [120720, 130560, 140400, 150240]
BOUNDS = [0] + _HALF + [160000] + [160000 + b for b in _HALF] + [320000]


def _spmm_body(h_hbm, src_hbm, attr_hbm, kidx_hbm, psrc_hbm, pattr_hbm,
               rdst_hbm, zeros_hbm, out_hbm, side_hbm,
               acc, cbuf, rows_v, src_sm, attr_sm, kidx_sm,
               psrc_v, pattr_v, rdst_v, side_v, sem):
    c = lax.axis_index("c")
    s = lax.axis_index("s")
    wid = c * NS + s

    @pl.when(s < 10)
    def _():
        pltpu.sync_copy(zeros_hbm.at[pl.ds(s * ZPT, ZPT)],
                        acc.at[pl.ds(s * ZPT, ZPT)])

    for p in range(4):
        pltpu.sync_copy(psrc_hbm.at[pl.ds((wid * 8 + p) * CB, CB)],
                        psrc_v[p])
    pltpu.sync_copy(pattr_hbm.at[pl.ds(wid * 8 * CB, 4 * CB)], pattr_v)
    pltpu.sync_copy(rdst_hbm.at[pl.ds(wid * 8, 8)], rdst_v)

    zero16 = jnp.zeros((16,), jnp.float32)

    def zrow(r, _):
        for j in range(H // 16):
            cbuf[r, pl.ds(j * 16, 16)] = zero16
        return 0

    lax.fori_loop(0, CBUF, zrow, 0)

    plsc.subcore_barrier()

    def block(ci, _):
        base = (wid * NB + ci) * CB
        pltpu.sync_copy(src_hbm.at[pl.ds(base, CB)], src_sm)
        pltpu.sync_copy(attr_hbm.at[pl.ds(base, CB)], attr_sm)
        pltpu.sync_copy(kidx_hbm.at[pl.ds(base, CB)], kidx_sm)
        pltpu.async_copy(h_hbm.at[src_sm], rows_v, sem).wait()

        def grp(g, _):
            avec = attr_sm[pl.ds(g * 16, 16)]
            kvec = kidx_sm[pl.ds(g * 16, 16)]
            for r16 in range(16):
                a = avec[r16]
                k = kvec[r16]
                r = g * 16 + r16
                for j in range(H // 16):
                    cbuf[k, pl.ds(j * 16, 16)] = (
                        cbuf[k, pl.ds(j * 16, 16)]
                        + rows_v[r, pl.ds(j * 16, 16)] * a)
            return 0

        lax.fori_loop(0, CB // 16, grp, 0)
        return 0

    lax.fori_loop(0, NB, block, 0)

    for b in range(CBUF // 128):
        pltpu.sync_copy(cbuf.at[pl.ds(b * 128, 128)],
                        acc.at[rdst_v.at[b]], add=True)

    for half in range(2):
        carry = (zero16,) * (H // 16)
        for p in (2 * half, 2 * half + 1):
            pltpu.async_copy(h_hbm.at[psrc_v[p]], rows_v, sem).wait()

            def fold16(g, cr, p=p):
                avec = pattr_v[pl.ds(p * CB + g * 16, 16)]
                for r16 in range(16):
                    a = avec[r16]
                    r = g * 16 + r16
                    cr = tuple(
                        cr[j] + rows_v[r, pl.ds(j * 16, 16)] * a
                        for j in range(H // 16))
                return cr

            carry = lax.fori_loop(0, CB // 16, fold16, carry)
        for j in range(H // 16):
            side_v[half, pl.ds(j * 16, 16)] = carry[j]

    pltpu.sync_copy(side_v, side_hbm.at[pl.ds(wid * 8, 8)])

    plsc.subcore_barrier()

    @pl.when(s < 10)
    def _():
        pltpu.sync_copy(acc.at[pl.ds(s * ZPT, ZPT)],
                        out_hbm.at[c, pl.ds(s * ZPT, ZPT)])


@functools.cache
def _spmm_kernel():
    return pl.kernel(
        _spmm_body,
        out_type=(jax.ShapeDtypeStruct((NC, SCH, H), jnp.float32),
                  jax.ShapeDtypeStruct((NW * 8, H), jnp.float32)),
        mesh=plsc.VectorSubcoreMesh(core_axis_name="c", subcore_axis_name="s",
                                    num_cores=NC, num_subcores=NS),
        scratch_types=[
            pltpu.VMEM_SHARED((SCH, H), jnp.float32),
            pltpu.VMEM((CBUF, H), jnp.float32),
            pltpu.VMEM((CB, H), jnp.float32),
            pltpu.VMEM((CB,), jnp.int32),
            pltpu.VMEM((CB,), jnp.float32),
            pltpu.VMEM((CB,), jnp.int32),
            [pltpu.VMEM((CB,), jnp.int32) for _ in range(4)],
            pltpu.VMEM((4 * CB,), jnp.float32),
            pltpu.VMEM((8, 128), jnp.int32),
            pltpu.VMEM((8, H), jnp.float32),
            pltpu.SemaphoreType.DMA,
        ],
    )


def _spmm(h, src, attr, kidx, psrc, pattr, rdst, zeros):
    return _spmm_kernel()(h, src, attr, kidx, psrc, pattr, rdst, zeros)


_RB = 1000
_GRID = N // _RB


def _mm1_body(x_ref, w_ref, b_ref, o_ref):
    o_ref[...] = (jnp.dot(x_ref[...], w_ref[...],
                          preferred_element_type=jnp.float32) + b_ref[...])


def _assemble(i, a0_ref, a1_ref, e_ref):
    zf = jnp.zeros((_RB, H), jnp.float32)
    a0 = jnp.where(i < SCH // _RB, a0_ref[...], zf)
    a1 = jnp.where(i >= BSC1 // _RB, a1_ref[...], zf)
    return (a0 + a1) + e_ref[...]


def _mm2_body(a0_ref, a1_ref, e_ref, w_ref, b_ref, o_ref):
    z = _assemble(pl.program_id(0), a0_ref, a1_ref, e_ref)
    o_ref[...] = jax.nn.relu(jnp.dot(z, w_ref[...],
                                     preferred_element_type=jnp.float32)
                             + b_ref[...])


def _mm2r_body(a0_ref, a1_ref, e_ref, w_ref, b_ref, r_ref, o_ref):
    z = _assemble(pl.program_id(0), a0_ref, a1_ref, e_ref)
    o_ref[...] = r_ref[...] + jax.nn.relu(
        jnp.dot(z, w_ref[...], preferred_element_type=jnp.float32)
        + b_ref[...])


def _mm1(x, w, b):
    return pl.pallas_call(
        _mm1_body,
        grid=(_GRID,),
        in_specs=[pl.BlockSpec((_RB, H), lambda i: (i, 0)),
                  pl.BlockSpec((H, H), lambda i: (0, 0)),
                  pl.BlockSpec((H,), lambda i: (0,))],
        out_specs=pl.BlockSpec((_RB, H), lambda i: (i, 0)),
        out_shape=jax.ShapeDtypeStruct((N, H), jnp.float32),
    )(x, w, b)


_a0_spec = pl.BlockSpec((_RB, H),
                        lambda i: (jnp.minimum(i, SCH // _RB - 1), 0))
_a1_spec = pl.BlockSpec((_RB, H),
                        lambda i: (jnp.maximum(i - BSC1 // _RB, 0), 0))


def _mm2(a0, a1, extra, w, b):
    return pl.pallas_call(
        _mm2_body,
        grid=(_GRID,),
        in_specs=[_a0_spec, _a1_spec,
                  pl.BlockSpec((_RB, H), lambda i: (i, 0)),
                  pl.BlockSpec((H, H), lambda i: (0, 0)),
                  pl.BlockSpec((H,), lambda i: (0,))],
        out_specs=pl.BlockSpec((_RB, H), lambda i: (i, 0)),
        out_shape=jax.ShapeDtypeStruct((N, H), jnp.float32),
    )(a0, a1, extra, w, b)


def _mm2r(a0, a1, extra, w, b, res):
    return pl.pallas_call(
        _mm2r_body,
        grid=(_GRID,),
        in_specs=[_a0_spec, _a1_spec,
                  pl.BlockSpec((_RB, H), lambda i: (i, 0)),
                  pl.BlockSpec((H, H), lambda i: (0, 0)),
                  pl.BlockSpec((H,), lambda i: (0,)),
                  pl.BlockSpec((_RB, H), lambda i: (i, 0))],
        out_specs=pl.BlockSpec((_RB, H), lambda i: (i, 0)),
        out_shape=jax.ShapeDtypeStruct((N, H), jnp.float32),
    )(a0, a1, extra, w, b, res)


def _pool_head_body(h_ref, batch_ref, wf1_ref, bf1_ref, g1_ref, be1_ref,
                    wf2_ref, bf2_ref, g2_ref, be2_ref, o_ref, acc):
    i = pl.program_id(0)
    onehot = (batch_ref[...].reshape(_RB, 1)
              == lax.broadcasted_iota(jnp.int32, (_RB, G), 1)
              ).astype(jnp.float32)
    part = lax.dot_general(onehot, h_ref[...], (((0,), (0,)), ((), ())),
                           preferred_element_type=jnp.float32,
                           precision=lax.Precision.HIGHEST)

    @pl.when(i == 0)
    def _():
        acc[...] = part

    @pl.when(i > 0)
    def _():
        acc[...] = acc[...] + part

    @pl.when(i == _GRID - 1)
    def _():
        def bn(z, g, b):
            mu = jnp.mean(z, axis=0)
            var = jnp.mean((z - mu) ** 2, axis=0)
            return (z - mu) / jnp.sqrt(var + 1e-5) * g + b

        z1 = jnp.dot(acc[...], wf1_ref[...],
                     preferred_element_type=jnp.float32) + bf1_ref[...]
        z = jax.nn.relu(bn(z1, g1_ref[...], be1_ref[...]))
        z2 = jnp.dot(z, wf2_ref[...],
                     preferred_element_type=jnp.float32) + bf2_ref[...]
        o_ref[...] = bn(z2, g2_ref[...], be2_ref[...])


def _pool_head(h, batch, wf1, bf1, g1, be1, wf2, bf2, g2, be2):
    full = lambda *shape: pl.BlockSpec(shape, lambda i: (0,) * len(shape))
    return pl.pallas_call(
        _pool_head_body,
        grid=(_GRID,),
        in_specs=[pl.BlockSpec((_RB, H), lambda i: (i, 0)),
                  pl.BlockSpec((1, 1, _RB), lambda i: (i, 0, 0)),
                  full(H, H), full(H), full(H), full(H),
                  full(H, C), full(C), full(C), full(C)],
        out_specs=pl.BlockSpec((G, C), lambda i: (0, 0)),
        out_shape=jax.ShapeDtypeStruct((G, C), jnp.float32),
        scratch_shapes=[pltpu.VMEM((G, H), jnp.float32)],
    )(h, batch.reshape(_GRID, 1, _RB), wf1, bf1, g1, be1, wf2, bf2, g2, be2)


def _prep_edges(src, dst, attr):
    perm = jnp.argsort(dst, stable=True)
    src_s = src[perm]
    dst_s = dst[perm]
    attr_s = attr[perm]

    bpos = jnp.asarray(BOUNDS[1:NW], jnp.int32)
    split = dst_s[bpos] == dst_s[bpos - 1]
    d_b = dst_s[bpos]
    rs = jnp.searchsorted(dst_s, d_b, side='left').astype(jnp.int32)
    re = jnp.searchsorted(dst_s, d_b, side='right').astype(jnp.int32)

    mark = jnp.zeros((E + 1,), jnp.int32)
    mark = mark.at[jnp.where(split, rs, E)].add(1)
    mark = mark.at[jnp.where(split, re, E)].add(-1)
    in_piece = jnp.cumsum(mark[:E]) > 0
    attr_main = jnp.where(in_piece, 0.0, attr_s)

    b_lo = jnp.asarray(BOUNDS[:NW], jnp.int32)
    b_hi = jnp.asarray(BOUNDS[1:], jnp.int32)
    k = jnp.arange(NB * CB, dtype=jnp.int32)
    gidx = b_lo[:, None] + k[None, :]
    valid = gidx < b_hi[:, None]
    gidx = jnp.where(valid, gidx, E)
    pad0_i = jnp.concatenate([src_s, jnp.zeros((1,), src_s.dtype)])
    pad0_d = jnp.concatenate([dst_s, jnp.zeros((1,), dst_s.dtype)])
    pad0_a = jnp.concatenate([attr_main, jnp.zeros((1,), attr_s.dtype)])
    src_m = pad0_i[gidx].reshape(-1)
    attr_m = pad0_a[gidx].reshape(-1)

    base_t = (dst_s[b_lo] // 8 * 8).astype(jnp.int32)
    kidx = pad0_d[gidx] - base_t[:, None]
    kidx = jnp.where(valid, jnp.clip(kidx, 0, CBUF - 1), 0)
    kidx_m = kidx.astype(jnp.int32).reshape(-1)

    base_sc = jnp.where(jnp.arange(NW) < NS, 0, BSC1).astype(jnp.int32)
    slot = jnp.arange(CBUF, dtype=jnp.int32)
    rloc = base_t[:, None] - base_sc[:, None] + slot[None, :]
    rloc = jnp.clip(rloc, 0, SCH - 1)
    rdst = jnp.zeros((NW, 8, 128), jnp.int32)
    rdst = rdst.at[:, :3, :].set(rloc.reshape(NW, 3, 128))
    rdst = rdst.reshape(NW * 8, 128)

    j = jnp.arange(PP, dtype=jnp.int32)
    head_len = jnp.where(split, re - bpos, 0)
    tail_len = jnp.where(split, bpos - rs, 0)
    hidx = jnp.where(j[None, :] < head_len[:, None], bpos[:, None] + j, E)
    tidx = jnp.where(j[None, :] < tail_len[:, None], rs[:, None] + j, E)
    empty = jnp.full((1, PP), E, jnp.int32)
    head_rows = jnp.concatenate([empty, hidx]).reshape(NW, 2, CB)
    tail_rows = jnp.concatenate([tidx, empty]).reshape(NW, 2, CB)
    pad_rows = jnp.full((NW, 4, CB), E, jnp.int32)
    pidx = jnp.concatenate([head_rows, tail_rows, pad_rows], axis=1)
    pad1_a = jnp.concatenate([attr_s, jnp.zeros((1,), attr_s.dtype)])
    psrc = pad0_i[pidx].reshape(-1)
    pattr = pad1_a[pidx].reshape(-1)

    d_vec = jnp.where(split, d_b, 0).astype(jnp.int32)
    return src_m, attr_m, kidx_m, psrc, pattr, rdst, d_vec


@jax.jit
def _run(x, batch, scatter_edge_index, scatter_edge_attr,
         W_in, b_in, Wc1, bc1, Wc2, bc2,
         Wf1, bf1, g1, be1, Wf2, bf2, g2, be2):
    src = scatter_edge_index[0].astype(jnp.int32)
    dst = scatter_edge_index[1].astype(jnp.int32)
    attr = scatter_edge_attr.astype(jnp.float32)
    (src_m, attr_m, kidx_m, psrc, pattr, rdst, d_vec) = _prep_edges(
        src, dst, attr)
    zeros = jnp.zeros((SCH, H), jnp.float32)
    batch = batch.astype(jnp.int32)

    def spmm_extra(hin):
        a, side = _spmm(hin, src_m, attr_m, kidx_m, psrc, pattr, rdst, zeros)
        side = side.reshape(NW, 8, H)
        combined = side[:NW - 1, 1] + side[1:, 0]
        extra = jnp.zeros((N, H), jnp.float32).at[d_vec].add(combined)
        return a[0], a[1], extra

    h = _mm1(x, W_in, b_in)
    for i in range(3):
        a0, a1, extra = spmm_extra(h)
        h1 = _mm2(a0, a1, extra, Wc1[i], bc1[i])
        b0, b1, extra2 = spmm_extra(h1)
        h = _mm2r(b0, b1, extra2, Wc2[i], bc2[i], h)
    return _pool_head(h, batch, Wf1, bf1, g1, be1, Wf2, bf2, g2, be2)


def kernel(x, edge_index, batch, scatter_edge_index, scatter_edge_attr,
           W_in, b_in, Wc1, bc1, Wc2, bc2,
           Wf1, bf1, g1, be1, Wf2, bf2, g2, be2):
    del edge_index
    return _run(x, batch, scatter_edge_index, scatter_edge_attr,
                W_in, b_in, Wc1, bc1, Wc2, bc2,
                Wf1, bf1, g1, be1, Wf2, bf2, g2, be2)


# speedup vs baseline: 1.0741x; 1.0337x over previous
import functools

import jax
import jax.numpy as jnp
from jax import lax
from jax.experimental import pallas as pl
from jax.experimental.pallas import tpu as pltpu
from jax.experimental.pallas import tpu_sc as plsc

N = 10000
E = 320000
H = 128
G = 64
C = 10

NC = 2
NS = 16
NW = NC * NS
CB = 64
NB = 160
CBUF = 384
SCH = 8000
BSC1 = 3000
ZPT = 800
PP = 128

_HALF = [10080 * k for k in range(1, 12)] + [120720, 130560, 140400, 150240]
BOUNDS = [0] + _HALF + [160000] + [160000 + b for b in _HALF] + [320000]


def _spmm_body(h_hbm, pk_hbm, attr_hbm, psrc_hbm, pattr_hbm,
               rdst_hbm, zeros_hbm, out_hbm, side_hbm,
               acc, cbuf, rows_v, pk_sm, attr_sm, src_sm,
               psrc_v, pattr_v, rdst_v, side_v, sem):
    c = lax.axis_index("c")
    s = lax.axis_index("s")
    wid = c * NS + s

    @pl.when(s < 10)
    def _():
        pltpu.sync_copy(zeros_hbm.at[pl.ds(s * ZPT, ZPT)],
                        acc.at[pl.ds(s * ZPT, ZPT)])

    for p in range(4):
        pltpu.sync_copy(psrc_hbm.at[pl.ds((wid * 8 + p) * CB, CB)],
                        psrc_v[p])
    pltpu.sync_copy(pattr_hbm.at[pl.ds(wid * 8 * CB, 4 * CB)], pattr_v)
    pltpu.sync_copy(rdst_hbm.at[pl.ds(wid * 8, 8)], rdst_v)

    zero16 = jnp.zeros((16,), jnp.float32)

    def zrow(r, _):
        for j in range(H // 16):
            cbuf[r, pl.ds(j * 16, 16)] = zero16
        return 0

    lax.fori_loop(0, CBUF, zrow, 0)

    plsc.subcore_barrier()

    def block(ci, _):
        pltpu.sync_copy(pk_hbm.at[pl.ds((wid * NB + ci) * 2 * CB, 2 * CB)],
                        pk_sm)
        pltpu.sync_copy(attr_hbm.at[pl.ds((wid * NB + ci) * CB, CB)],
                        attr_sm)
        for g in range(CB // 16):
            src_sm[pl.ds(g * 16, 16)] = pk_sm[pl.ds(g * 16, 16)]
        pltpu.async_copy(h_hbm.at[src_sm], rows_v, sem).wait()

        def grp(g, _):
            avec = attr_sm[pl.ds(g * 16, 16)]
            kvec = pk_sm[pl.ds(CB + g * 16, 16)]
            for r16 in range(16):
                a = avec[r16]
                k = kvec[r16]
                r = g * 16 + r16
                for j in range(H // 16):
                    cbuf[k, pl.ds(j * 16, 16)] = (
                        cbuf[k, pl.ds(j * 16, 16)]
                        + rows_v[r, pl.ds(j * 16, 16)] * a)
            return 0

        lax.fori_loop(0, CB // 16, grp, 0)
        return 0

    lax.fori_loop(0, NB, block, 0)

    for b in range(CBUF // 128):
        pltpu.sync_copy(cbuf.at[pl.ds(b * 128, 128)],
                        acc.at[rdst_v.at[b]], add=True)

    for half in range(2):
        carry = (zero16,) * (H // 16)
        for p in (2 * half, 2 * half + 1):
            pltpu.async_copy(h_hbm.at[psrc_v[p]], rows_v, sem).wait()

            def fold16(g, cr, p=p):
                avec = pattr_v[pl.ds(p * CB + g * 16, 16)]
                for r16 in range(16):
                    a = avec[r16]
                    r = g * 16 + r16
                    cr = tuple(
                        cr[j] + rows_v[r, pl.ds(j * 16, 16)] * a
                        for j in range(H // 16))
                return cr

            carry = lax.fori_loop(0, CB // 16, fold16, carry)
        for j in range(H // 16):
            side_v[half, pl.ds(j * 16, 16)] = carry[j]

    pltpu.sync_copy(side_v, side_hbm.at[pl.ds(wid * 8, 8)])

    plsc.subcore_barrier()

    @pl.when(s < 10)
    def _():
        pltpu.sync_copy(acc.at[pl.ds(s * ZPT, ZPT)],
                        out_hbm.at[c, pl.ds(s * ZPT, ZPT)])


@functools.cache
def _spmm_kernel():
    return pl.kernel(
        _spmm_body,
        out_type=(jax.ShapeDtypeStruct((NC, SCH, H), jnp.float32),
                  jax.ShapeDtypeStruct((NW * 8, H), jnp.float32)),
        mesh=plsc.VectorSubcoreMesh(core_axis_name="c", subcore_axis_name="s",
                                    num_cores=NC, num_subcores=NS),
        scratch_types=[
            pltpu.VMEM_SHARED((SCH, H), jnp.float32),
            pltpu.VMEM((CBUF, H), jnp.float32),
            pltpu.VMEM((CB, H), jnp.float32),
            pltpu.VMEM((2 * CB,), jnp.int32),
            pltpu.VMEM((CB,), jnp.float32),
            pltpu.VMEM((CB,), jnp.int32),
            [pltpu.VMEM((CB,), jnp.int32) for _ in range(4)],
            pltpu.VMEM((4 * CB,), jnp.float32),
            pltpu.VMEM((8, 128), jnp.int32),
            pltpu.VMEM((8, H), jnp.float32),
            pltpu.SemaphoreType.DMA,
        ],
    )


def _spmm(h, pk, attr_f, psrc, pattr, rdst, zeros):
    return _spmm_kernel()(h, pk, attr_f, psrc, pattr, rdst, zeros)


_RB = 1000
_GRID = N // _RB


def _mm1_body(x_ref, w_ref, b_ref, o_ref):
    o_ref[...] = (jnp.dot(x_ref[...], w_ref[...],
                          preferred_element_type=jnp.float32) + b_ref[...])


def _assemble(i, a0_ref, a1_ref, e_ref):
    zf = jnp.zeros((_RB, H), jnp.float32)
    a0 = jnp.where(i < SCH // _RB, a0_ref[...], zf)
    a1 = jnp.where(i >= BSC1 // _RB, a1_ref[...], zf)
    return (a0 + a1) + e_ref[...]


def _mm2_body(a0_ref, a1_ref, e_ref, w_ref, b_ref, o_ref):
    z = _assemble(pl.program_id(0), a0_ref, a1_ref, e_ref)
    o_ref[...] = jax.nn.relu(jnp.dot(z, w_ref[...],
                                     preferred_element_type=jnp.float32)
                             + b_ref[...])


def _mm2r_body(a0_ref, a1_ref, e_ref, w_ref, b_ref, r_ref, o_ref):
    z = _assemble(pl.program_id(0), a0_ref, a1_ref, e_ref)
    o_ref[...] = r_ref[...] + jax.nn.relu(
        jnp.dot(z, w_ref[...], preferred_element_type=jnp.float32)
        + b_ref[...])


def _mm1(x, w, b):
    return pl.pallas_call(
        _mm1_body,
        grid=(_GRID,),
        in_specs=[pl.BlockSpec((_RB, H), lambda i: (i, 0)),
                  pl.BlockSpec((H, H), lambda i: (0, 0)),
                  pl.BlockSpec((H,), lambda i: (0,))],
        out_specs=pl.BlockSpec((_RB, H), lambda i: (i, 0)),
        out_shape=jax.ShapeDtypeStruct((N, H), jnp.float32),
    )(x, w, b)


_a0_spec = pl.BlockSpec((_RB, H),
                        lambda i: (jnp.minimum(i, SCH // _RB - 1), 0))
_a1_spec = pl.BlockSpec((_RB, H),
                        lambda i: (jnp.maximum(i - BSC1 // _RB, 0), 0))


def _mm2(a0, a1, extra, w, b):
    return pl.pallas_call(
        _mm2_body,
        grid=(_GRID,),
        in_specs=[_a0_spec, _a1_spec,
                  pl.BlockSpec((_RB, H), lambda i: (i, 0)),
                  pl.BlockSpec((H, H), lambda i: (0, 0)),
                  pl.BlockSpec((H,), lambda i: (0,))],
        out_specs=pl.BlockSpec((_RB, H), lambda i: (i, 0)),
        out_shape=jax.ShapeDtypeStruct((N, H), jnp.float32),
    )(a0, a1, extra, w, b)


def _mm2r(a0, a1, extra, w, b, res):
    return pl.pallas_call(
        _mm2r_body,
        grid=(_GRID,),
        in_specs=[_a0_spec, _a1_spec,
                  pl.BlockSpec((_RB, H), lambda i: (i, 0)),
                  pl.BlockSpec((H, H), lambda i: (0, 0)),
                  pl.BlockSpec((H,), lambda i: (0,)),
                  pl.BlockSpec((_RB, H), lambda i: (i, 0))],
        out_specs=pl.BlockSpec((_RB, H), lambda i: (i, 0)),
        out_shape=jax.ShapeDtypeStruct((N, H), jnp.float32),
    )(a0, a1, extra, w, b, res)


def _pool_head_body(h_ref, batch_ref, wf1_ref, bf1_ref, g1_ref, be1_ref,
                    wf2_ref, bf2_ref, g2_ref, be2_ref, o_ref, acc):
    i = pl.program_id(0)
    onehot = (batch_ref[...].reshape(_RB, 1)
              == lax.broadcasted_iota(jnp.int32, (_RB, G), 1)
              ).astype(jnp.float32)
    part = lax.dot_general(onehot, h_ref[...], (((0,), (0,)), ((), ())),
                           preferred_element_type=jnp.float32,
                           precision=lax.Precision.HIGHEST)

    @pl.when(i == 0)
    def _():
        acc[...] = part

    @pl.when(i > 0)
    def _():
        acc[...] = acc[...] + part

    @pl.when(i == _GRID - 1)
    def _():
        def bn(z, g, b):
            mu = jnp.mean(z, axis=0)
            var = jnp.mean((z - mu) ** 2, axis=0)
            return (z - mu) / jnp.sqrt(var + 1e-5) * g + b

        z1 = jnp.dot(acc[...], wf1_ref[...],
                     preferred_element_type=jnp.float32) + bf1_ref[...]
        z = jax.nn.relu(bn(z1, g1_ref[...], be1_ref[...]))
        z2 = jnp.dot(z, wf2_ref[...],
                     preferred_element_type=jnp.float32) + bf2_ref[...]
        o_ref[...] = bn(z2, g2_ref[...], be2_ref[...])


def _pool_head(h, batch, wf1, bf1, g1, be1, wf2, bf2, g2, be2):
    full = lambda *shape: pl.BlockSpec(shape, lambda i: (0,) * len(shape))
    return pl.pallas_call(
        _pool_head_body,
        grid=(_GRID,),
        in_specs=[pl.BlockSpec((_RB, H), lambda i: (i, 0)),
                  pl.BlockSpec((1, 1, _RB), lambda i: (i, 0, 0)),
                  full(H, H), full(H), full(H), full(H),
                  full(H, C), full(C), full(C), full(C)],
        out_specs=pl.BlockSpec((G, C), lambda i: (0, 0)),
        out_shape=jax.ShapeDtypeStruct((G, C), jnp.float32),
        scratch_shapes=[pltpu.VMEM((G, H), jnp.float32)],
    )(h, batch.reshape(_GRID, 1, _RB), wf1, bf1, g1, be1, wf2, bf2, g2, be2)


def _prep_edges(src, dst, attr):
    perm = jnp.argsort(dst, stable=True)
    src_s = src[perm]
    dst_s = dst[perm]
    attr_s = attr[perm]

    bpos = jnp.asarray(BOUNDS[1:NW], jnp.int32)
    split = dst_s[bpos] == dst_s[bpos - 1]
    d_b = dst_s[bpos]
    rs = jnp.searchsorted(dst_s, d_b, side='left').astype(jnp.int32)
    re = jnp.searchsorted(dst_s, d_b, side='right').astype(jnp.int32)

    mark = jnp.zeros((E + 1,), jnp.int32)
    mark = mark.at[jnp.where(split, rs, E)].add(1)
    mark = mark.at[jnp.where(split, re, E)].add(-1)
    in_piece = jnp.cumsum(mark[:E]) > 0
    attr_main = jnp.where(in_piece, 0.0, attr_s)

    b_lo = jnp.asarray(BOUNDS[:NW], jnp.int32)
    b_hi = jnp.asarray(BOUNDS[1:], jnp.int32)
    k = jnp.arange(NB * CB, dtype=jnp.int32)
    gidx = b_lo[:, None] + k[None, :]
    valid = gidx < b_hi[:, None]
    gidx = jnp.where(valid, gidx, E)
    pad0_i = jnp.concatenate([src_s, jnp.zeros((1,), src_s.dtype)])
    pad0_d = jnp.concatenate([dst_s, jnp.zeros((1,), dst_s.dtype)])
    pad0_a = jnp.concatenate([attr_main, jnp.zeros((1,), attr_s.dtype)])
    src_m = pad0_i[gidx].reshape(NW * NB, CB)
    attr_m = pad0_a[gidx].reshape(NW * NB, CB)

    base_t = (dst_s[b_lo] // 8 * 8).astype(jnp.int32)
    kidx = pad0_d[gidx] - base_t[:, None]
    kidx = jnp.where(valid, jnp.clip(kidx, 0, CBUF - 1), 0)
    kidx_m = kidx.astype(jnp.int32).reshape(NW * NB, CB)
    pk = jnp.stack([src_m.astype(jnp.int32), kidx_m],
                   axis=1).reshape(-1)
    attr_f = attr_m.reshape(-1)

    base_sc = jnp.where(jnp.arange(NW) < NS, 0, BSC1).astype(jnp.int32)
    slot = jnp.arange(CBUF, dtype=jnp.int32)
    rloc = base_t[:, None] - base_sc[:, None] + slot[None, :]
    rloc = jnp.clip(rloc, 0, SCH - 1)
    rdst = jnp.zeros((NW, 8, 128), jnp.int32)
    rdst = rdst.at[:, :3, :].set(rloc.reshape(NW, 3, 128))
    rdst = rdst.reshape(NW * 8, 128)

    j = jnp.arange(PP, dtype=jnp.int32)
    head_len = jnp.where(split, re - bpos, 0)
    tail_len = jnp.where(split, bpos - rs, 0)
    hidx = jnp.where(j[None, :] < head_len[:, None], bpos[:, None] + j, E)
    tidx = jnp.where(j[None, :] < tail_len[:, None], rs[:, None] + j, E)
    empty = jnp.full((1, PP), E, jnp.int32)
    head_rows = jnp.concatenate([empty, hidx]).reshape(NW, 2, CB)
    tail_rows = jnp.concatenate([tidx, empty]).reshape(NW, 2, CB)
    pad_rows = jnp.full((NW, 4, CB), E, jnp.int32)
    pidx = jnp.concatenate([head_rows, tail_rows, pad_rows], axis=1)
    pad1_a = jnp.concatenate([attr_s, jnp.zeros((1,), attr_s.dtype)])
    psrc = pad0_i[pidx].reshape(-1)
    pattr = pad1_a[pidx].reshape(-1)

    d_vec = jnp.where(split, d_b, 0).astype(jnp.int32)
    return pk, attr_f, psrc, pattr, rdst, d_vec


@jax.jit
def _run(x, batch, scatter_edge_index, scatter_edge_attr,
         W_in, b_in, Wc1, bc1, Wc2, bc2,
         Wf1, bf1, g1, be1, Wf2, bf2, g2, be2):
    src = scatter_edge_index[0].astype(jnp.int32)
    dst = scatter_edge_index[1].astype(jnp.int32)
    attr = scatter_edge_attr.astype(jnp.float32)
    pk, attr_f, psrc, pattr, rdst, d_vec = _prep_edges(src, dst, attr)
    zeros = jnp.zeros((SCH, H), jnp.float32)
    batch = batch.astype(jnp.int32)

    def spmm_extra(hin):
        a, side = _spmm(hin, pk, attr_f, psrc, pattr, rdst, zeros)
        side = side.reshape(NW, 8, H)
        combined = side[:NW - 1, 1] + side[1:, 0]
        extra = jnp.zeros((N, H), jnp.float32).at[d_vec].add(combined)
        return a[0], a[1], extra

    h = _mm1(x, W_in, b_in)
    for i in range(3):
        a0, a1, extra = spmm_extra(h)
        h1 = _mm2(a0, a1, extra, Wc1[i], bc1[i])
        b0, b1, extra2 = spmm_extra(h1)
        h = _mm2r(b0, b1, extra2, Wc2[i], bc2[i], h)
    return _pool_head(h, batch, Wf1, bf1, g1, be1, Wf2, bf2, g2, be2)


def kernel(x, edge_index, batch, scatter_edge_index, scatter_edge_attr,
           W_in, b_in, Wc1, bc1, Wc2, bc2,
           Wf1, bf1, g1, be1, Wf2, bf2, g2, be2):
    del edge_index
    return _run(x, batch, scatter_edge_index, scatter_edge_attr,
                W_in, b_in, Wc1, bc1, Wc2, bc2,
                Wf1, bf1, g1, be1, Wf2, bf2, g2, be2)
